# baseline (speedup 1.0000x reference)
import jax
import jax.numpy as jnp
from jax import lax
from jax.experimental import pallas as pl
from jax.experimental.pallas import tpu as pltpu
from jax.experimental.pallas import tpu_sc as plsc

_SCALE = 64.0
_MARGIN = 0.35
_B = 1024
_V = 100000
_LOG2E = 1.4426950408889634
_C2 = _SCALE * _LOG2E

_NC = 2
_NS = 16
_NW = _NC * _NS
_RPW = _B // _NW
_TRW = _RPW // 8

_CW = 1408
_CPC = 17
_K = 4
_SPAN = _CPC * _CW
_CSC = _K * _SPAN
_UNR = 4
_ILOOP = _CW // (16 * _UNR)

_BR = 32
_CB0 = _CSC // _CW
_NCB = (_V - _CSC + _CW - 1) // _CW


def _sc_slice_body(cos_hbm, label_hbm, s_hbm, t_hbm,
                   lab_v, t_v, s_v, *bufsems, c0):
    bufs = bufsems[:2 * _TRW]
    sems = bufsems[2 * _TRW:4 * _TRW]
    wid = lax.axis_index("s") * _NC + lax.axis_index("c")
    base = wid * _RPW
    iota16 = lax.iota(jnp.int32, 16)
    zeros16 = jnp.zeros((16,), jnp.float32)

    pltpu.sync_copy(label_hbm.at[pl.ds(base, _RPW)], lab_v)
    for j in range(_RPW):
        s_v[pl.ds(j * 16, 16)] = zeros16
    for j in range(_RPW // 16):
        t_v[pl.ds(j * 16, 16)] = zeros16

    lab_cc, lab_go, lab_lane = [], [], []
    for rl in range(_RPW):
        grp = lab_v[pl.ds((rl // 16) * 16, 16)]
        rel = grp[rl % 16] - c0
        o_lab = rel % _CW
        go = (o_lab // 16) * 16
        lab_cc.append(rel // _CW)
        lab_go.append(go)
        lab_lane.append(o_lab - go)

    def issue(cc, tr, p):
        pltpu.async_copy(
            cos_hbm.at[pl.ds(base + tr * 8, 8), pl.ds(cc * _CW, _CW)],
            bufs[tr * 2 + p], sems[tr * 2 + p])

    for tr in range(_TRW):
        issue(0, tr, 0)
        issue(1, tr, 1)

    def outer(i, carry):
        for p in range(2):
            cc = 2 * i + p
            for tr in range(_TRW):
                nb = tr * 2 + p

                @pl.when(cc < _CPC)
                def _(cc=cc, tr=tr, nb=nb, p=p):
                    pltpu.make_async_copy(
                        cos_hbm.at[pl.ds(0, 8), pl.ds(0, _CW)], bufs[nb],
                        sems[nb]).wait()
                    for r in range(8):
                        def vbody(k, ab, nb=nb, r=r):
                            a0, a1 = ab
                            off = k * (16 * _UNR)
                            for u in range(_UNR):
                                v = bufs[nb][r, pl.ds(off + u * 16, 16)]
                                e = jnp.exp(v * _SCALE)
                                if u % 2 == 0:
                                    a0 = a0 + e
                                else:
                                    a1 = a1 + e
                            return (a0, a1)

                        a0, a1 = lax.fori_loop(0, _ILOOP, vbody,
                                               (zeros16, zeros16))
                        rl = tr * 8 + r
                        sl = pl.ds(rl * 16, 16)
                        s_v[sl] = s_v[sl] + (a0 + a1)

                        @pl.when(cc == lab_cc[rl])
                        def _(nb=nb, r=r, rl=rl):
                            vec = bufs[nb][r, pl.ds(lab_go[rl], 16)]
                            idxv = jnp.zeros((16,), jnp.int32) + lab_lane[rl]
                            tv = lax.gather(
                                vec, idxv[:, None],
                                lax.GatherDimensionNumbers(
                                    offset_dims=(), collapsed_slice_dims=(0,),
                                    start_index_map=(0,)),
                                slice_sizes=(1,),
                                mode=lax.GatherScatterMode.PROMISE_IN_BOUNDS)
                            tsl = pl.ds((rl // 16) * 16, 16)
                            t_v[tsl] = t_v[tsl] + jnp.where(
                                iota16 == (rl % 16), tv, 0.0)

                    @pl.when(cc + 2 < _CPC)
                    def _(cc=cc, tr=tr, p=p):
                        issue(cc + 2, tr, p)
        return carry

    lax.fori_loop(0, (_CPC + 1) // 2, outer, jnp.int32(0))

    pltpu.sync_copy(s_v, s_hbm.at[pl.ds(base * 16, _RPW * 16)])
    pltpu.sync_copy(t_v, t_hbm.at[pl.ds(base, _RPW)])


def _sc_slice(cosine_slice, label, c0):
    import functools
    mesh = plsc.VectorSubcoreMesh(core_axis_name="c", subcore_axis_name="s")
    return pl.kernel(
        functools.partial(_sc_slice_body, c0=c0),
        out_type=[jax.ShapeDtypeStruct((_B * 16,), jnp.float32),
                  jax.ShapeDtypeStruct((_B,), jnp.float32)],
        mesh=mesh,
        scratch_types=(
            [pltpu.VMEM((_RPW,), jnp.int32),
             pltpu.VMEM((_RPW,), jnp.float32),
             pltpu.VMEM((_RPW * 16,), jnp.float32)]
            + [pltpu.VMEM((8, _CW), jnp.float32) for _ in range(2 * _TRW)]
            + [pltpu.SemaphoreType.DMA for _ in range(2 * _TRW)]
        ),
    )(cosine_slice, label.astype(jnp.int32))


def _tc_sweep_body(lab_ref, cos_ref, s_out, t_out):
    j = pl.program_id(1)
    x = cos_ref[...]
    col = lax.broadcasted_iota(jnp.int32, x.shape, 1) + _CSC + j * _CW
    valid = col < _V
    spart = jnp.sum(jnp.where(valid, jnp.exp2(x * _C2), 0.0),
                    axis=1, keepdims=True)
    lab = lab_ref[...]
    tpart = jnp.sum(jnp.where(col == lab, x, 0.0), axis=1, keepdims=True)
    zero = jnp.zeros_like(spart)
    s_out[...] = jnp.where(j == 0, zero, s_out[...]) + spart
    t_out[...] = jnp.where(j == 0, zero, t_out[...]) + tpart


def _tc_sweep(label, cosine):
    return pl.pallas_call(
        _tc_sweep_body,
        grid=(_B // _BR, _NCB),
        in_specs=[
            pl.BlockSpec((_BR, 1), lambda i, j: (i, 0)),
            pl.BlockSpec((_BR, _CW), lambda i, j: (i, _CB0 + j)),
        ],
        out_specs=[
            pl.BlockSpec((_BR, 1), lambda i, j: (i, 0)),
            pl.BlockSpec((_BR, 1), lambda i, j: (i, 0)),
        ],
        out_shape=[jax.ShapeDtypeStruct((_B, 1), jnp.float32),
                   jax.ShapeDtypeStruct((_B, 1), jnp.float32)],
    )(label.astype(jnp.int32).reshape(_B, 1), cosine)


def _tc_combine_body(stc_ref, ttc_ref, *refs):
    srefs = refs[:_K]
    trefs = refs[_K:2 * _K]
    out_ref = refs[2 * _K]
    s = stc_ref[...]
    t = ttc_ref[...]
    for k in range(_K):
        s = s + jnp.sum(srefs[k][...], axis=1, keepdims=True)
        t = t + trefs[k][...]
    t64 = t * _SCALE
    delta = _SCALE * _MARGIN
    sp = s - jnp.exp(t64) * (1.0 - jnp.exp(jnp.float32(-delta)))
    nll = jnp.log(sp) - t64 + delta
    out_ref[...] = jnp.sum(nll, keepdims=True) * (1.0 / _B)


def _tc_combine(s_tc, t_tc, s_list, t_list):
    out = pl.pallas_call(
        _tc_combine_body,
        in_specs=(
            [pl.BlockSpec((_B, 1), lambda: (0, 0)),
             pl.BlockSpec((_B, 1), lambda: (0, 0))]
            + [pl.BlockSpec((_B, 16), lambda: (0, 0)) for _ in range(_K)]
            + [pl.BlockSpec((_B, 1), lambda: (0, 0)) for _ in range(_K)]
        ),
        out_specs=pl.BlockSpec((1, 1), lambda: (0, 0)),
        out_shape=jax.ShapeDtypeStruct((1, 1), jnp.float32),
    )(s_tc, t_tc,
      *[s.reshape(_B, 16) for s in s_list],
      *[t.reshape(_B, 1) for t in t_list])
    return out[0, 0]


def kernel(cosine, label):
    s_list, t_list = [], []
    for k in range(_K):
        c0 = k * _SPAN
        s_k, t_k = _sc_slice(cosine[:, c0:c0 + _SPAN], label, c0)
        s_list.append(s_k)
        t_list.append(t_k)
    s_tc, t_tc = _tc_sweep(label, cosine)
    return _tc_combine(s_tc, t_tc, s_list, t_list)


# speedup vs baseline: 1.5582x; 1.5582x over previous
import jax
import jax.numpy as jnp
from jax import lax
from jax.experimental import pallas as pl
from jax.experimental.pallas import tpu as pltpu
from jax.experimental.pallas import tpu_sc as plsc

_SCALE = 64.0
_MARGIN = 0.35
_B = 1024
_V = 100000
_VMAIN = 99968
_LOG2E = 1.4426950408889634
_C2 = _SCALE * _LOG2E

_NC = 2
_NS = 16
_NW = _NC * _NS
_RPW = _B // _NW
_TRW = _RPW // 8

_CW = 1408
_CPT = 58
_CSC = _CPT * _CW
_UNR = 4
_ILOOP = _CW // (16 * _UNR)

_BR = 32
_NCB = (_V - _CSC + _CW - 1) // _CW


def _sc_main_body(cos_hbm, label_hbm, s_hbm, t_hbm,
                  lab_v, t_v, s_v, *bufsems):
    bufs = bufsems[:_TRW]
    sems = bufsems[_TRW:2 * _TRW]
    wid = lax.axis_index("s") * _NC + lax.axis_index("c")
    base = wid * _RPW
    iota16 = lax.iota(jnp.int32, 16)
    zeros16 = jnp.zeros((16,), jnp.float32)

    pltpu.sync_copy(label_hbm.at[pl.ds(base, _RPW)], lab_v)
    for j in range(_RPW):
        s_v[pl.ds(j * 16, 16)] = zeros16
    for j in range(_RPW // 16):
        t_v[pl.ds(j * 16, 16)] = zeros16

    lab_cc, lab_go, lab_lane = [], [], []
    for rl in range(_RPW):
        grp = lab_v[pl.ds((rl // 16) * 16, 16)]
        lab_s = grp[rl % 16]
        o_lab = lab_s % _CW
        go = (o_lab // 16) * 16
        lab_cc.append(lab_s // _CW)
        lab_go.append(go)
        lab_lane.append(o_lab - go)

    def issue(cc, tr):
        pltpu.async_copy(
            cos_hbm.at[pl.ds(base + tr * 8, 8), pl.ds(cc * _CW, _CW)],
            bufs[tr], sems[tr])

    for tr in range(_TRW):
        issue(0, tr)

    def outer(cc, carry):
        for tr in range(_TRW):
            pltpu.make_async_copy(
                cos_hbm.at[pl.ds(0, 8), pl.ds(0, _CW)], bufs[tr],
                sems[tr]).wait()
            for r in range(8):
                def vbody(k, ab, tr=tr, r=r):
                    a0, a1 = ab
                    off = k * (16 * _UNR)
                    for u in range(_UNR):
                        v = bufs[tr][r, pl.ds(off + u * 16, 16)]
                        e = jnp.exp(v * _SCALE)
                        if u % 2 == 0:
                            a0 = a0 + e
                        else:
                            a1 = a1 + e
                    return (a0, a1)

                a0, a1 = lax.fori_loop(0, _ILOOP, vbody, (zeros16, zeros16))
                rl = tr * 8 + r
                sl = pl.ds(rl * 16, 16)
                s_v[sl] = s_v[sl] + (a0 + a1)

                @pl.when(cc == lab_cc[rl])
                def _(tr=tr, r=r, rl=rl):
                    vec = bufs[tr][r, pl.ds(lab_go[rl], 16)]
                    idxv = jnp.zeros((16,), jnp.int32) + lab_lane[rl]
                    tv = lax.gather(
                        vec, idxv[:, None],
                        lax.GatherDimensionNumbers(
                            offset_dims=(), collapsed_slice_dims=(0,),
                            start_index_map=(0,)),
                        slice_sizes=(1,),
                        mode=lax.GatherScatterMode.PROMISE_IN_BOUNDS)
                    tsl = pl.ds((rl // 16) * 16, 16)
                    t_v[tsl] = t_v[tsl] + jnp.where(iota16 == (rl % 16),
                                                    tv, 0.0)

            @pl.when(cc + 1 < _CPT)
            def _(tr=tr):
                issue(cc + 1, tr)
        return carry

    lax.fori_loop(0, _CPT, outer, jnp.int32(0))

    pltpu.sync_copy(s_v, s_hbm.at[pl.ds(base * 16, _RPW * 16)])
    pltpu.sync_copy(t_v, t_hbm.at[pl.ds(base, _RPW)])


def _sc_main(cosine, label):
    mesh = plsc.VectorSubcoreMesh(core_axis_name="c", subcore_axis_name="s")
    return pl.kernel(
        _sc_main_body,
        out_type=[jax.ShapeDtypeStruct((_B * 16,), jnp.float32),
                  jax.ShapeDtypeStruct((_B,), jnp.float32)],
        mesh=mesh,
        scratch_types=(
            [pltpu.VMEM((_RPW,), jnp.int32),
             pltpu.VMEM((_RPW,), jnp.float32),
             pltpu.VMEM((_RPW * 16,), jnp.float32)]
            + [pltpu.VMEM((8, _CW), jnp.float32) for _ in range(_TRW)]
            + [pltpu.SemaphoreType.DMA for _ in range(_TRW)]
        ),
    )(cosine, label.astype(jnp.int32))


def _tc_sweep_body(lab_ref, cos_ref, s_out, t_out):
    j = pl.program_id(1)
    x = cos_ref[...]
    col = lax.broadcasted_iota(jnp.int32, x.shape, 1) + _CSC + j * _CW
    valid = col < _V
    spart = jnp.sum(jnp.where(valid, jnp.exp2(x * _C2), 0.0),
                    axis=1, keepdims=True)
    lab = lab_ref[...]
    tpart = jnp.sum(jnp.where(col == lab, x, 0.0), axis=1, keepdims=True)
    zero = jnp.zeros_like(spart)
    s_out[...] = jnp.where(j == 0, zero, s_out[...]) + spart
    t_out[...] = jnp.where(j == 0, zero, t_out[...]) + tpart


def _tc_sweep(label, cosine):
    return pl.pallas_call(
        _tc_sweep_body,
        grid=(_B // _BR, _NCB),
        in_specs=[
            pl.BlockSpec((_BR, 1), lambda i, j: (i, 0)),
            pl.BlockSpec((_BR, _CW), lambda i, j: (i, _CPT + j)),
        ],
        out_specs=[
            pl.BlockSpec((_BR, 1), lambda i, j: (i, 0)),
            pl.BlockSpec((_BR, 1), lambda i, j: (i, 0)),
        ],
        out_shape=[jax.ShapeDtypeStruct((_B, 1), jnp.float32),
                   jax.ShapeDtypeStruct((_B, 1), jnp.float32)],
    )(label.astype(jnp.int32).reshape(_B, 1), cosine)


def _tc_combine_body(s_ref, t_ref, stc_ref, ttc_ref, out_ref):
    s = jnp.sum(s_ref[...], axis=1, keepdims=True) + stc_ref[...]
    t64 = (t_ref[...] + ttc_ref[...]) * _SCALE
    delta = _SCALE * _MARGIN
    sp = s - jnp.exp(t64) * (1.0 - jnp.exp(jnp.float32(-delta)))
    nll = jnp.log(sp) - t64 + delta
    out_ref[...] = jnp.sum(nll, keepdims=True) * (1.0 / _B)


def _tc_combine(s, t, s_tc, t_tc):
    out = pl.pallas_call(
        _tc_combine_body,
        in_specs=[
            pl.BlockSpec((_B, 16), lambda: (0, 0)),
            pl.BlockSpec((_B, 1), lambda: (0, 0)),
            pl.BlockSpec((_B, 1), lambda: (0, 0)),
            pl.BlockSpec((_B, 1), lambda: (0, 0)),
        ],
        out_specs=pl.BlockSpec((1, 1), lambda: (0, 0)),
        out_shape=jax.ShapeDtypeStruct((1, 1), jnp.float32),
    )(s.reshape(_B, 16), t.reshape(_B, 1), s_tc, t_tc)
    return out[0, 0]


def kernel(cosine, label):
    s, t = _sc_main(cosine, label)
    s_tc, t_tc = _tc_sweep(label, cosine)
    return _tc_combine(s, t, s_tc, t_tc)
